# baseline (speedup 1.0000x reference)
import functools

import jax
import jax.numpy as jnp
from jax import lax
from jax.experimental import pallas as pl
from jax.experimental.pallas import tpu as pltpu
from jax.experimental.pallas import tpu_sc as plsc

B = 16
H = 16
N = 577
NUM_REL = 2212
TBL = NUM_REL * H
L = 16
NW = 32

NR = 584
NRG = NR // 8
NC = 640
CHUNK = 8 * N
CLOAD = 4640
IDXP = (NRG - 1) * CHUNK + CLOAD
NTASK = NRG * 2
CJ = 37
SLAB = 8 * 8 * NC

RB = 96


@functools.cache
def _sc_gather_bias_fn():
    @functools.partial(
        pl.kernel,
        mesh=plsc.VectorSubcoreMesh(core_axis_name="c", subcore_axis_name="s"),
        out_type=jax.ShapeDtypeStruct((NR, H, NC), jnp.float32),
        scratch_types=[
            pltpu.VMEM((H, NUM_REL), jnp.float32),
            pltpu.VMEM((CLOAD,), jnp.int32),
            pltpu.VMEM((2, 8, 8, NC), jnp.float32),
            pltpu.SemaphoreType.DMA,
        ],
        compiler_params=pltpu.CompilerParams(needs_layout_passes=False),
    )
    def _sc_gather_bias(tbl_hbm, idx_hbm, out_hbm, tbl_v, idx_v, buf_v, sem):
        wid = lax.axis_index("s") * 2 + lax.axis_index("c")
        start = 4 * wid + (9 * wid) // 16
        end = 4 * (wid + 1) + (9 * (wid + 1)) // 16
        pltpu.sync_copy(tbl_hbm, tbl_v)

        def _task(t, carry):
            rg = t // 2
            hp = (t % 2) * 8

            @pl.when((t == start) | (t % 2 == 0))
            def _load_idx():
                off = pl.multiple_of(rg * CHUNK, 8)
                pltpu.sync_copy(idx_hbm.at[pl.ds(off, CLOAD)], idx_v)

            p = t % 2

            @pl.when(t >= start + 2)
            def _drain():
                pltpu.make_async_copy(
                    out_hbm.at[pl.ds(0, 8), pl.ds(0, 8), :], buf_v.at[0], sem
                ).wait()

            def _row(r, c1):
                base = r * N

                @plsc.parallel_loop(0, CJ, unroll=4)
                def _col(j):
                    v = idx_v[pl.ds(base + j * L, L)]
                    zeros = v * 0
                    for hh in range(8):
                        hv = zeros + (hp + hh)
                        g = plsc.load_gather(tbl_v, [hv, v])
                        buf_v[p, r, hh, pl.ds(j * L, L)] = g

                return c1

            lax.fori_loop(0, 8, _row, 0)
            pltpu.async_copy(
                buf_v.at[p],
                out_hbm.at[pl.ds(rg * 8, 8), pl.ds(hp, 8), :],
                sem,
            )
            return carry

        lax.fori_loop(start, end, _task, 0)

        pltpu.make_async_copy(out_hbm.at[pl.ds(0, 8), pl.ds(0, 8), :], buf_v.at[0], sem).wait()
        pltpu.make_async_copy(out_hbm.at[pl.ds(0, 8), pl.ds(0, 8), :], buf_v.at[1], sem).wait()

    return _sc_gather_bias


def _add_body(a_ref, b_ref, o_ref):
    o_ref[...] = a_ref[...] + b_ref[:, :, :N]


def kernel(attn, relative_position_bias_table, relative_position_index):
    tbl_t = relative_position_bias_table.T
    idx_flat = relative_position_index.reshape(-1).astype(jnp.int32)
    idx_pad = jnp.pad(idx_flat, (0, IDXP - N * N))

    bias = _sc_gather_bias_fn()(tbl_t, idx_pad)

    attn_t = attn.transpose(0, 2, 1, 3)
    nrb = pl.cdiv(N, RB)
    out_t = pl.pallas_call(
        _add_body,
        grid=(nrb, B // 2),
        in_specs=[
            pl.BlockSpec((2, RB, H, N), lambda r, b: (b, r, 0, 0)),
            pl.BlockSpec((RB, H, NC), lambda r, b: (r, 0, 0)),
        ],
        out_specs=pl.BlockSpec((2, RB, H, N), lambda r, b: (b, r, 0, 0)),
        out_shape=jax.ShapeDtypeStruct((B, N, H, N), jnp.float32),
    )(attn_t, bias)
    return out_t.transpose(0, 2, 1, 3)


# speedup vs baseline: 1.0270x; 1.0270x over previous
import functools

import jax
import jax.numpy as jnp
from jax import lax
from jax.experimental import pallas as pl
from jax.experimental.pallas import tpu as pltpu
from jax.experimental.pallas import tpu_sc as plsc

B = 16
H = 16
N = 577
NUM_REL = 2212
TBL = NUM_REL * H
L = 16
NW = 32

NR = 584
NRG = NR // 8
NC = 640
CHUNK = 8 * N
CLOAD = 4640
IDXP = (NRG - 1) * CHUNK + CLOAD
NTASK = NRG * 2
CJ = 37
SLAB = 8 * 8 * NC

RB = 195


@functools.cache
def _sc_gather_bias_fn():
    @functools.partial(
        pl.kernel,
        mesh=plsc.VectorSubcoreMesh(core_axis_name="c", subcore_axis_name="s"),
        out_type=jax.ShapeDtypeStruct((NR, H, NC), jnp.float32),
        scratch_types=[
            pltpu.VMEM((H, NUM_REL), jnp.float32),
            pltpu.VMEM((CLOAD,), jnp.int32),
            pltpu.VMEM((2, 8, 8, NC), jnp.float32),
            pltpu.SemaphoreType.DMA,
        ],
        compiler_params=pltpu.CompilerParams(needs_layout_passes=False),
    )
    def _sc_gather_bias(tbl_hbm, idx_hbm, out_hbm, tbl_v, idx_v, buf_v, sem):
        wid = lax.axis_index("s") * 2 + lax.axis_index("c")
        start = 4 * wid + (9 * wid) // 16
        end = 4 * (wid + 1) + (9 * (wid + 1)) // 16
        pltpu.sync_copy(tbl_hbm, tbl_v)

        def _task(t, carry):
            rg = t // 2
            hp = (t % 2) * 8

            @pl.when((t == start) | (t % 2 == 0))
            def _load_idx():
                off = pl.multiple_of(rg * CHUNK, 8)
                pltpu.sync_copy(idx_hbm.at[pl.ds(off, CLOAD)], idx_v)

            p = t % 2

            @pl.when(t >= start + 2)
            def _drain():
                pltpu.make_async_copy(
                    out_hbm.at[pl.ds(0, 8), pl.ds(0, 8), :], buf_v.at[0], sem
                ).wait()

            def _row(r, c1):
                base = r * N

                @plsc.parallel_loop(0, CJ, unroll=4)
                def _col(j):
                    v = idx_v[pl.ds(base + j * L, L)]
                    zeros = v * 0
                    for hh in range(8):
                        hv = zeros + (hp + hh)
                        g = plsc.load_gather(tbl_v, [hv, v])
                        buf_v[p, r, hh, pl.ds(j * L, L)] = g

                return c1

            lax.fori_loop(0, 8, _row, 0)
            pltpu.async_copy(
                buf_v.at[p],
                out_hbm.at[pl.ds(rg * 8, 8), pl.ds(hp, 8), :],
                sem,
            )
            return carry

        lax.fori_loop(start, end, _task, 0)

        pltpu.make_async_copy(out_hbm.at[pl.ds(0, 8), pl.ds(0, 8), :], buf_v.at[0], sem).wait()
        pltpu.make_async_copy(out_hbm.at[pl.ds(0, 8), pl.ds(0, 8), :], buf_v.at[1], sem).wait()

    return _sc_gather_bias


def _add_body(a_ref, b_ref, o_ref):
    o_ref[...] = a_ref[...] + b_ref[:, :, :N]


def kernel(attn, relative_position_bias_table, relative_position_index):
    tbl_t = relative_position_bias_table.T
    idx_flat = relative_position_index.reshape(-1).astype(jnp.int32)
    idx_pad = jnp.pad(idx_flat, (0, IDXP - N * N))

    bias = _sc_gather_bias_fn()(tbl_t, idx_pad)

    attn_t = attn.transpose(0, 2, 1, 3)
    nrb = pl.cdiv(N, RB)
    out_t = pl.pallas_call(
        _add_body,
        grid=(nrb, B),
        in_specs=[
            pl.BlockSpec((1, RB, H, N), lambda r, b: (b, r, 0, 0)),
            pl.BlockSpec((RB, H, NC), lambda r, b: (r, 0, 0)),
        ],
        out_specs=pl.BlockSpec((1, RB, H, N), lambda r, b: (b, r, 0, 0)),
        out_shape=jax.ShapeDtypeStruct((B, N, H, N), jnp.float32),
    )(attn_t, bias)
    return out_t.transpose(0, 2, 1, 3)
